# baseline (speedup 1.0000x reference)
import functools

import jax
import jax.numpy as jnp
from jax import lax
from jax.experimental import pallas as pl
from jax.experimental.pallas import tpu as pltpu
from jax.experimental.pallas import tpu_sc as plsc

B = 1024
D = 16

_info = plsc.get_sparse_core_info()
_NC, _NS = _info.num_cores, _info.num_subcores
_NW = _NC * _NS
_BPW = B // _NW

_sc_mesh = plsc.VectorSubcoreMesh(core_axis_name="c", subcore_axis_name="s")


@functools.partial(
    pl.kernel,
    mesh=_sc_mesh,
    compiler_params=pltpu.CompilerParams(use_tc_tiling_on_sc=False),
    out_type=[
        jax.ShapeDtypeStruct((B, D), jnp.float32),
        jax.ShapeDtypeStruct((B, D), jnp.float32),
        jax.ShapeDtypeStruct((B,), jnp.float32),
    ],
    scratch_types=[
        pltpu.VMEM((_BPW,), jnp.int32),
        pltpu.VMEM((_BPW,), jnp.int32),
        pltpu.VMEM((_BPW, D), jnp.float32),
        pltpu.VMEM((_BPW, D), jnp.float32),
        pltpu.VMEM((_BPW,), jnp.float32),
        pltpu.VMEM((_BPW,), jnp.float32),
        pltpu.SemaphoreType.DMA,
        pltpu.SemaphoreType.DMA,
        pltpu.SemaphoreType.DMA,
        pltpu.SemaphoreType.DMA,
    ],
)
def _sc_gather(uid_hbm, iid_hbm, ut_hbm, it_hbm, ub_hbm, ib_hbm,
               urep_hbm, irep_hbm, bias_hbm,
               uidx_v, iidx_v, urows_v, irows_v, ub_v, ib_v,
               sem_u, sem_i, sem_ub, sem_ib):
    wid = lax.axis_index("s") * _NC + lax.axis_index("c")
    base = wid * _BPW
    pltpu.sync_copy(uid_hbm.at[pl.ds(base, _BPW)], uidx_v)
    pltpu.sync_copy(iid_hbm.at[pl.ds(base, _BPW)], iidx_v)
    cu = pltpu.async_copy(ut_hbm.at[uidx_v], urows_v, sem_u)
    ci = pltpu.async_copy(it_hbm.at[iidx_v], irows_v, sem_i)
    cub = pltpu.async_copy(ub_hbm.at[uidx_v], ub_v, sem_ub)
    cib = pltpu.async_copy(ib_hbm.at[iidx_v], ib_v, sem_ib)
    cu.wait()
    ci.wait()
    cub.wait()
    cib.wait()
    for t in range(_BPW // 16):
        s = pl.ds(t * 16, 16)
        ub_v[s] = ub_v[s] + ib_v[s]
    pltpu.sync_copy(urows_v, urep_hbm.at[pl.ds(base, _BPW)])
    pltpu.sync_copy(irows_v, irep_hbm.at[pl.ds(base, _BPW)])
    pltpu.sync_copy(ub_v, bias_hbm.at[pl.ds(base, _BPW)])


_G = 4
_BR = B // _G


def _tc_body(irep_ref, urep_ref, bias_ref, out_ref):
    out_ref[...] = lax.dot_general(
        irep_ref[...], urep_ref[...],
        dimension_numbers=(((1,), (1,)), ((), ())),
        preferred_element_type=jnp.float32,
    ) + bias_ref[...]


_tc_matmul = pl.pallas_call(
    _tc_body,
    grid=(_G,),
    in_specs=[
        pl.BlockSpec((_BR, D), lambda i: (i, 0)),
        pl.BlockSpec((B, D), lambda i: (0, 0)),
        pl.BlockSpec((1, B), lambda i: (0, 0)),
    ],
    out_specs=pl.BlockSpec((_BR, B), lambda i: (i, 0)),
    out_shape=jax.ShapeDtypeStruct((B, B), jnp.float32),
)


def kernel(user_id, item_id, user_table, item_table, user_bias_table, item_bias_table):
    ub_flat = user_bias_table.reshape(-1)
    ib_flat = item_bias_table.reshape(-1)
    urep, irep, bias = _sc_gather(
        user_id.astype(jnp.int32), item_id.astype(jnp.int32),
        user_table, item_table, ub_flat, ib_flat)
    out = _tc_matmul(irep, urep, bias.reshape(1, B))
    return out[:, :, None]
